# Initial kernel scaffold; baseline (speedup 1.0000x reference)
import jax
import jax.numpy as jnp
from jax.experimental import pallas as pl


def kernel(logits, action):
    raise NotImplementedError("write your pallas kernel here")


# speedup vs baseline: 1.2330x; 1.2330x over previous
import jax
import jax.numpy as jnp
from jax import lax
from jax.experimental import pallas as pl
from jax.experimental.pallas import tpu as pltpu
from jax.experimental.pallas import tpu_sc as plsc

B = 16384
C = 25
H = 10
NC, NS, L = 2, 16, 16
NW = NC * NS
RW = B // NW
NG = RW // L
TRI_OFF = (0, 3, 6, 9, 12)
DUO_OFF = (15, 17, 19, 21, 23)
LN2 = 0.6931471805599453


def _log_1to3(s):
    v = (s - 2.0) / (s + 2.0)
    v2 = v * v
    p = v2 * (1.0 / 11.0) + (1.0 / 9.0)
    p = v2 * p + (1.0 / 7.0)
    p = v2 * p + (1.0 / 5.0)
    p = v2 * p + (1.0 / 3.0)
    p = v2 * p + 1.0
    return LN2 + 2.0 * v * p


def _body(lg_hbm, ac_hbm, out_hbm, lg_v, ac_v, out_v):
    wid = lax.axis_index("s") * NC + lax.axis_index("c")
    pltpu.sync_copy(lg_hbm.at[pl.ds(wid * (RW * C), RW * C)], lg_v)
    pltpu.sync_copy(ac_hbm.at[pl.ds(wid * (RW * H), RW * H)], ac_v)
    iota = lax.iota(jnp.int32, L)
    iota_c = iota * C
    iota_h = iota * H
    iota_2 = iota * 2

    def group(g, carry):
        lbase = iota_c + g * (L * C)
        abase = iota_h + g * (L * H)
        lp = jnp.zeros((L,), jnp.float32)
        ent = jnp.zeros((L,), jnp.float32)
        for h, off in enumerate(TRI_OFF):
            x0 = plsc.load_gather(lg_v, [lbase + off])
            x1 = plsc.load_gather(lg_v, [lbase + (off + 1)])
            x2 = plsc.load_gather(lg_v, [lbase + (off + 2)])
            m = jnp.maximum(jnp.maximum(x0, x1), x2)
            e0 = jnp.exp(x0 - m)
            e1 = jnp.exp(x1 - m)
            e2 = jnp.exp(x2 - m)
            s = e0 + e1 + e2
            lse = m + _log_1to3(s)
            w = (e0 * x0 + e1 * x1 + e2 * x2) / s
            ent = ent + (lse - w)
            a = plsc.load_gather(ac_v, [abase + h])
            xa = jnp.where(a == 0, x0, jnp.where(a == 1, x1, x2))
            lp = lp + (xa - lse)
        for h, off in enumerate(DUO_OFF):
            x0 = plsc.load_gather(lg_v, [lbase + off])
            x1 = plsc.load_gather(lg_v, [lbase + (off + 1)])
            m = jnp.maximum(x0, x1)
            e0 = jnp.exp(x0 - m)
            e1 = jnp.exp(x1 - m)
            s = e0 + e1
            lse = m + _log_1to3(s)
            w = (e0 * x0 + e1 * x1) / s
            ent = ent + (lse - w)
            a = plsc.load_gather(ac_v, [abase + (h + 5)])
            xa = jnp.where(a == 0, x0, x1)
            lp = lp + (xa - lse)
        obase = iota_2 + g * (L * 2)
        plsc.store_scatter(out_v, [obase], lp)
        plsc.store_scatter(out_v, [obase + 1], ent)
        return carry

    lax.fori_loop(0, NG, group, 0)
    pltpu.sync_copy(out_v, out_hbm.at[pl.ds(wid * (RW * 2), RW * 2)])


def kernel(logits, action):
    lg = logits.reshape(B * C)
    ac = action.reshape(B * H)
    run = pl.kernel(
        _body,
        out_type=jax.ShapeDtypeStruct((B * 2,), jnp.float32),
        mesh=plsc.VectorSubcoreMesh(
            core_axis_name="c", subcore_axis_name="s",
            num_cores=NC, num_subcores=NS,
        ),
        scratch_types=[
            pltpu.VMEM((RW * C,), jnp.float32),
            pltpu.VMEM((RW * H,), jnp.int32),
            pltpu.VMEM((RW * 2,), jnp.float32),
        ],
        compiler_params=pltpu.CompilerParams(needs_layout_passes=False),
    )
    return run(lg, ac).reshape(B, 2)
